# Initial kernel scaffold; baseline (speedup 1.0000x reference)
import jax
import jax.numpy as jnp
from jax.experimental import pallas as pl


def kernel(x, adj, edge_weights, W1, b1, W2, b2):
    raise NotImplementedError("write your pallas kernel here")


# speedup vs baseline: 7.3668x; 7.3668x over previous
import jax
import jax.numpy as jnp
from jax import lax
from jax.experimental import pallas as pl
from jax.experimental.pallas import tpu as pltpu
from jax.experimental.pallas import tpu_sc as plsc

_NC = 2
_NS = 16
_NW = _NC * _NS
_L = 16
_C = 80
_WBR = 128


def _round_up(a, b):
    return (a + b - 1) // b * b


def _bcast_lane(v16, j):
    idx = jnp.full((_L, 1), j, jnp.int32)
    dn = lax.GatherDimensionNumbers(
        offset_dims=(), collapsed_slice_dims=(0,), start_index_map=(0,))
    return lax.gather(v16, idx, dn, (1,),
                      mode=lax.GatherScatterMode.PROMISE_IN_BOUNDS)


def _make_deg_kernel(epad, npad):
    ew = epad // _NW
    nchunk = ew // _C
    rpt = npad // _NS

    def body(col_hbm, w_hbm, out_hbm, colv, wv, zbuf, acc):
        cid = lax.axis_index("c")
        sid = lax.axis_index("s")
        wid = sid * _NC + cid
        z = jnp.zeros((_L,), jnp.float32)

        def zero_body(i, carry):
            zbuf[pl.ds(i * _L, _L)] = z
            return carry

        lax.fori_loop(0, rpt // _L, zero_body, 0)
        pltpu.sync_copy(zbuf, acc.at[pl.ds(sid * rpt, rpt)])
        plsc.subcore_barrier()

        def chunk(i, carry):
            base = wid * ew + i * _C
            pltpu.sync_copy(col_hbm.at[pl.ds(base, _C)], colv)
            pltpu.sync_copy(w_hbm.at[pl.ds(base, _C)], wv)
            pltpu.sync_copy(wv, acc.at[colv], add=True)
            return carry

        lax.fori_loop(0, nchunk, chunk, 0)
        plsc.subcore_barrier()
        pltpu.sync_copy(acc.at[pl.ds(sid * rpt, rpt)], zbuf)
        pltpu.sync_copy(zbuf, out_hbm.at[cid, pl.ds(sid * rpt, rpt)])

    return pl.kernel(
        body,
        out_type=jax.ShapeDtypeStruct((_NC, npad), jnp.float32),
        mesh=plsc.VectorSubcoreMesh(
            core_axis_name="c", subcore_axis_name="s",
            num_cores=_NC, num_subcores=_NS),
        scratch_types=[
            pltpu.VMEM((_C,), jnp.int32),
            pltpu.VMEM((_C,), jnp.float32),
            pltpu.VMEM((rpt,), jnp.float32),
            pltpu.VMEM_SHARED((npad,), jnp.float32),
        ],
    )


def _make_msg_kernel(epad, npad, d):
    ew = epad // _NW
    nchunk = ew // _C
    rpt = npad // _NS
    nwb = rpt // _WBR

    def body(h_hbm, row_hbm, col_hbm, w_hbm, z_hbm, out_hbm,
             rowv, colv, wv, msgs, wb, acc, sem):
        cid = lax.axis_index("c")
        sid = lax.axis_index("s")
        wid = sid * _NC + cid

        pltpu.sync_copy(z_hbm, wb)
        for k in range(nwb):
            pltpu.sync_copy(wb, acc.at[pl.ds(sid * rpt + k * _WBR, _WBR)])
        plsc.subcore_barrier()

        def chunk(i, carry):
            base = wid * ew + i * _C
            pltpu.sync_copy(row_hbm.at[pl.ds(base, _C)], rowv)
            pltpu.sync_copy(col_hbm.at[pl.ds(base, _C)], colv)
            pltpu.sync_copy(w_hbm.at[pl.ds(base, _C)], wv)
            pltpu.async_copy(h_hbm.at[rowv], msgs, sem).wait()

            def scale(g, c2):
                wvv = wv[pl.ds(g * _L, _L)]
                for j in range(_L):
                    s = _bcast_lane(wvv, j)
                    e2 = g * _L + j
                    for dch in range(d // _L):
                        sl = pl.ds(dch * _L, _L)
                        msgs[e2, sl] = msgs[e2, sl] * s
                return c2

            lax.fori_loop(0, _C // _L, scale, 0)
            pltpu.sync_copy(msgs, acc.at[colv], add=True)
            return carry

        lax.fori_loop(0, nchunk, chunk, 0)
        plsc.subcore_barrier()
        for k in range(nwb):
            off = sid * rpt + k * _WBR
            pltpu.sync_copy(acc.at[pl.ds(off, _WBR)], wb)
            pltpu.sync_copy(wb, out_hbm.at[cid, pl.ds(off, _WBR)])

    return pl.kernel(
        body,
        out_type=jax.ShapeDtypeStruct((_NC, npad, d), jnp.float32),
        mesh=plsc.VectorSubcoreMesh(
            core_axis_name="c", subcore_axis_name="s",
            num_cores=_NC, num_subcores=_NS),
        scratch_types=[
            pltpu.VMEM((_C,), jnp.int32),
            pltpu.VMEM((_C,), jnp.int32),
            pltpu.VMEM((_C,), jnp.float32),
            pltpu.VMEM((_C, d), jnp.float32),
            pltpu.VMEM((_WBR, d), jnp.float32),
            pltpu.VMEM_SHARED((npad, d), jnp.float32),
            pltpu.SemaphoreType.DMA,
        ],
    )


def _tc1_body(deg_ref, x_ref, w_ref, dis_ref, h_ref):
    n = x_ref.shape[0]
    deg = deg_ref[0] + deg_ref[1]
    dis = jnp.where(deg > 0.0, lax.rsqrt(deg), 0.0)
    dis_ref[...] = dis
    h = jnp.dot(x_ref[...], w_ref[...], precision=lax.Precision.HIGHEST)
    h_ref[...] = h * dis[:n]


def _tc2_body(s_ref, dis_ref, b_ref, w_ref, out_ref):
    n = out_ref.shape[0]
    s = s_ref[0, :n] + s_ref[1, :n]
    dis = dis_ref[:n]
    g = jnp.maximum(s * dis + b_ref[...][None, :], 0.0)
    out_ref[...] = jnp.dot(
        g, w_ref[...], precision=lax.Precision.HIGHEST) * dis


def _tc3_body(s_ref, dis_ref, b_ref, x_ref, out_ref):
    n = x_ref.shape[0]
    s = s_ref[0, :n] + s_ref[1, :n]
    out_ref[...] = (
        jnp.maximum(s * dis_ref[:n] + b_ref[...][None, :], 0.0)
        + x_ref[...])


def kernel(x, adj, edge_weights, W1, b1, W2, b2):
    n, d = x.shape
    e = edge_weights.shape[0]
    npad = _round_up(n, _NS * _WBR)
    epad = _round_up(e, _NW * _C)

    row = adj[0].astype(jnp.int32)
    col = adj[1].astype(jnp.int32)
    w = edge_weights.astype(jnp.float32)
    if epad > e:
        pz = epad - e
        row = jnp.concatenate([row, jnp.zeros((pz,), jnp.int32)])
        col = jnp.concatenate([col, jnp.zeros((pz,), jnp.int32)])
        w = jnp.concatenate([w, jnp.zeros((pz,), jnp.float32)])
    zeros = jnp.zeros((_WBR, d), jnp.float32)

    f32 = jnp.float32
    deg2 = _make_deg_kernel(epad, npad)(col, w)
    deg2 = deg2.reshape(_NC, npad, 1)

    dis, h1 = pl.pallas_call(
        _tc1_body,
        out_shape=[jax.ShapeDtypeStruct((npad, 1), f32),
                   jax.ShapeDtypeStruct((n, d), f32)],
    )(deg2, x, W1)

    msg = _make_msg_kernel(epad, npad, d)
    s1 = msg(h1, row, col, w, zeros)

    h2 = pl.pallas_call(
        _tc2_body,
        out_shape=jax.ShapeDtypeStruct((n, d), f32),
    )(s1, dis, b1, W2)

    s2 = msg(h2, row, col, w, zeros)

    out = pl.pallas_call(
        _tc3_body,
        out_shape=jax.ShapeDtypeStruct((n, d), f32),
    )(s2, dis, b2, x)

    return out, adj, edge_weights
